# baseline (device time: 32423 ns/iter reference)
import functools

import jax
import jax.numpy as jnp
from jax import lax
from jax.experimental import pallas as pl
from jax.experimental.pallas import tpu as pltpu

N_DEV = 16
T = 512
D = 512
V_PER = 4096
ROWS = T // N_DEV


def kernel(ids, E):
    ids2 = ids.reshape(T, 1)

    def body(ids_ref, e_ref, out_ref, send_buf, acc_ref, gbuf,
             send1, recv1, send2, recv2):
        my = lax.axis_index("i")

        barrier_sem = pltpu.get_barrier_semaphore()
        for j in range(N_DEV):
            @pl.when(my != j)
            def _(j=j):
                pl.semaphore_signal(
                    barrier_sem, inc=1,
                    device_id=(j,), device_id_type=pl.DeviceIdType.MESH,
                )
        pl.semaphore_wait(barrier_sem, N_DEV - 1)

        cols = lax.broadcasted_iota(jnp.int32, (T, V_PER), 1)
        local = ids_ref[:, :] - my * V_PER
        onehot = (cols == local).astype(jnp.bfloat16)
        e_bf = e_ref[:, :].astype(jnp.bfloat16)
        partial = jnp.dot(onehot, e_bf, preferred_element_type=jnp.float32)
        send_buf[:, :] = partial.astype(jnp.bfloat16)

        acc_ref[pl.ds(my, 1)] = send_buf[pl.ds(my * ROWS, ROWS), :].reshape(
            1, ROWS, D)

        for j in range(N_DEV):
            @pl.when(my != j)
            def _(j=j):
                rdma = pltpu.make_async_remote_copy(
                    src_ref=send_buf.at[pl.ds(j * ROWS, ROWS), :],
                    dst_ref=acc_ref.at[my],
                    send_sem=send1.at[j],
                    recv_sem=recv1.at[my],
                    device_id=(j,),
                    device_id_type=pl.DeviceIdType.MESH,
                )
                rdma.start()

        for s in range(N_DEV):
            @pl.when(my != s)
            def _(s=s):
                desc = pltpu.make_async_remote_copy(
                    src_ref=send_buf.at[pl.ds(0, ROWS), :],
                    dst_ref=acc_ref.at[s],
                    send_sem=send1.at[s],
                    recv_sem=recv1.at[s],
                    device_id=(s,),
                    device_id_type=pl.DeviceIdType.MESH,
                )
                desc.wait_recv()

        red = jnp.sum(acc_ref[:, :, :].astype(jnp.float32), axis=0)
        gbuf[pl.ds(my, 1)] = red.astype(jnp.bfloat16).reshape(1, ROWS, D)

        for j in range(N_DEV):
            @pl.when(my != j)
            def _(j=j):
                rdma = pltpu.make_async_remote_copy(
                    src_ref=gbuf.at[my],
                    dst_ref=gbuf.at[my],
                    send_sem=send2.at[j],
                    recv_sem=recv2.at[my],
                    device_id=(j,),
                    device_id_type=pl.DeviceIdType.MESH,
                )
                rdma.start()

        for s in range(N_DEV):
            @pl.when(my != s)
            def _(s=s):
                desc = pltpu.make_async_remote_copy(
                    src_ref=gbuf.at[s],
                    dst_ref=gbuf.at[s],
                    send_sem=send2.at[s],
                    recv_sem=recv2.at[s],
                    device_id=(s,),
                    device_id_type=pl.DeviceIdType.MESH,
                )
                desc.wait_recv()

        out_ref[:, :] = gbuf[:, :, :].reshape(T, D).astype(jnp.float32)

        for j in range(N_DEV):
            @pl.when(my != j)
            def _(j=j):
                d1 = pltpu.make_async_remote_copy(
                    src_ref=send_buf.at[pl.ds(j * ROWS, ROWS), :],
                    dst_ref=acc_ref.at[my],
                    send_sem=send1.at[j],
                    recv_sem=recv1.at[my],
                    device_id=(j,),
                    device_id_type=pl.DeviceIdType.MESH,
                )
                d1.wait_send()
                d2 = pltpu.make_async_remote_copy(
                    src_ref=gbuf.at[my],
                    dst_ref=gbuf.at[my],
                    send_sem=send2.at[j],
                    recv_sem=recv2.at[my],
                    device_id=(j,),
                    device_id_type=pl.DeviceIdType.MESH,
                )
                d2.wait_send()

        @functools.partial(pl.run_scoped, sem=pltpu.SemaphoreType.REGULAR)
        def _(sem):
            for j in range(N_DEV):
                @pl.when(my != j)
                def _(j=j):
                    pl.semaphore_signal(
                        sem, inc=1,
                        device_id=(j,), device_id_type=pl.DeviceIdType.MESH,
                    )
            pl.semaphore_wait(sem, N_DEV - 1)

    return pl.pallas_call(
        body,
        out_shape=jax.ShapeDtypeStruct((T, D), jnp.float32),
        in_specs=[
            pl.BlockSpec(memory_space=pltpu.VMEM),
            pl.BlockSpec(memory_space=pltpu.VMEM),
        ],
        out_specs=pl.BlockSpec(memory_space=pltpu.VMEM),
        scratch_shapes=[
            pltpu.VMEM((T, D), jnp.bfloat16),
            pltpu.VMEM((N_DEV, ROWS, D), jnp.bfloat16),
            pltpu.VMEM((N_DEV, ROWS, D), jnp.bfloat16),
            pltpu.SemaphoreType.DMA((N_DEV,)),
            pltpu.SemaphoreType.DMA((N_DEV,)),
            pltpu.SemaphoreType.DMA((N_DEV,)),
            pltpu.SemaphoreType.DMA((N_DEV,)),
        ],
        compiler_params=pltpu.CompilerParams(collective_id=0),
    )(ids2, E)


# device time: 25972 ns/iter; 1.2484x vs baseline; 1.2484x over previous
import jax
import jax.numpy as jnp
from jax import lax
from jax.experimental import pallas as pl
from jax.experimental.pallas import tpu as pltpu

N_DEV = 16
T = 512
D = 512
V_PER = 4096
ROWS = T // N_DEV
CHUNK = 128
N_CHUNK = T // CHUNK
DEST_PER_CHUNK = CHUNK // ROWS


def kernel(ids, E):
    ids2 = ids.reshape(T, 1)

    def body(ids_ref, e_ref, out_ref, send_buf, acc_ref, gbuf,
             send1, recv1, send2, recv2):
        my = lax.axis_index("i")

        barrier_sem = pltpu.get_barrier_semaphore()
        for j in range(N_DEV):
            @pl.when(my != j)
            def _(j=j):
                pl.semaphore_signal(
                    barrier_sem, inc=1,
                    device_id=(j,), device_id_type=pl.DeviceIdType.MESH,
                )

        e_bf = e_ref[:, :].astype(jnp.bfloat16)
        base = my * V_PER

        for c in range(N_CHUNK):
            lo = c * CHUNK
            cols = lax.broadcasted_iota(jnp.int32, (CHUNK, V_PER), 1)
            local = ids_ref[pl.ds(lo, CHUNK), :] - base
            onehot = (cols == local).astype(jnp.bfloat16)
            part = jnp.dot(onehot, e_bf,
                           preferred_element_type=jnp.float32)
            send_buf[pl.ds(lo, CHUNK), :] = part.astype(jnp.bfloat16)

            if c == 0:
                pl.semaphore_wait(barrier_sem, N_DEV - 1)

            for j in range(c * DEST_PER_CHUNK, (c + 1) * DEST_PER_CHUNK):
                @pl.when(my != j)
                def _(j=j):
                    rdma = pltpu.make_async_remote_copy(
                        src_ref=send_buf.at[pl.ds(j * ROWS, ROWS), :],
                        dst_ref=acc_ref.at[my],
                        send_sem=send1.at[j],
                        recv_sem=recv1.at[my],
                        device_id=(j,),
                        device_id_type=pl.DeviceIdType.MESH,
                    )
                    rdma.start()

                @pl.when(my == j)
                def _(j=j):
                    acc_ref[pl.ds(my, 1)] = send_buf[
                        pl.ds(my * ROWS, ROWS), :].reshape(1, ROWS, D)

        for s in range(N_DEV):
            @pl.when(my != s)
            def _(s=s):
                desc = pltpu.make_async_remote_copy(
                    src_ref=send_buf.at[pl.ds(0, ROWS), :],
                    dst_ref=acc_ref.at[s],
                    send_sem=send1.at[s],
                    recv_sem=recv1.at[s],
                    device_id=(s,),
                    device_id_type=pl.DeviceIdType.MESH,
                )
                desc.wait_recv()

        red = jnp.sum(acc_ref[:, :, :].astype(jnp.float32), axis=0)
        gbuf[pl.ds(my, 1)] = red.astype(jnp.bfloat16).reshape(1, ROWS, D)

        for j in range(N_DEV):
            @pl.when(my != j)
            def _(j=j):
                rdma = pltpu.make_async_remote_copy(
                    src_ref=gbuf.at[my],
                    dst_ref=gbuf.at[my],
                    send_sem=send2.at[j],
                    recv_sem=recv2.at[my],
                    device_id=(j,),
                    device_id_type=pl.DeviceIdType.MESH,
                )
                rdma.start()

        for s in range(N_DEV):
            @pl.when(my != s)
            def _(s=s):
                desc = pltpu.make_async_remote_copy(
                    src_ref=gbuf.at[s],
                    dst_ref=gbuf.at[s],
                    send_sem=send2.at[s],
                    recv_sem=recv2.at[s],
                    device_id=(s,),
                    device_id_type=pl.DeviceIdType.MESH,
                )
                desc.wait_recv()

        out_ref[:, :] = gbuf[:, :, :].reshape(T, D).astype(jnp.float32)

        for j in range(N_DEV):
            @pl.when(my != j)
            def _(j=j):
                d1 = pltpu.make_async_remote_copy(
                    src_ref=send_buf.at[pl.ds(j * ROWS, ROWS), :],
                    dst_ref=acc_ref.at[my],
                    send_sem=send1.at[j],
                    recv_sem=recv1.at[my],
                    device_id=(j,),
                    device_id_type=pl.DeviceIdType.MESH,
                )
                d1.wait_send()
                d2 = pltpu.make_async_remote_copy(
                    src_ref=gbuf.at[my],
                    dst_ref=gbuf.at[my],
                    send_sem=send2.at[j],
                    recv_sem=recv2.at[my],
                    device_id=(j,),
                    device_id_type=pl.DeviceIdType.MESH,
                )
                d2.wait_send()

    return pl.pallas_call(
        body,
        out_shape=jax.ShapeDtypeStruct((T, D), jnp.float32),
        in_specs=[
            pl.BlockSpec(memory_space=pltpu.VMEM),
            pl.BlockSpec(memory_space=pltpu.VMEM),
        ],
        out_specs=pl.BlockSpec(memory_space=pltpu.VMEM),
        scratch_shapes=[
            pltpu.VMEM((T, D), jnp.bfloat16),
            pltpu.VMEM((N_DEV, ROWS, D), jnp.bfloat16),
            pltpu.VMEM((N_DEV, ROWS, D), jnp.bfloat16),
            pltpu.SemaphoreType.DMA((N_DEV,)),
            pltpu.SemaphoreType.DMA((N_DEV,)),
            pltpu.SemaphoreType.DMA((N_DEV,)),
            pltpu.SemaphoreType.DMA((N_DEV,)),
        ],
        compiler_params=pltpu.CompilerParams(collective_id=0),
    )(ids2, E)
